# baseline (speedup 1.0000x reference)
import jax
import jax.numpy as jnp
from jax import lax
from jax.experimental import pallas as pl
from jax.experimental.pallas import tpu as pltpu
from jax.experimental.pallas import tpu_sc as plsc

_B = 16384
_C = 1000
_NC, _NS, _L = 2, 16, 16
_NW = _NC * _NS
_ROWS_PER_TILE = _B // _NW
_CHUNK = 32
_NCHUNK = _ROWS_PER_TILE // _CHUNK
_GROUPS = _CHUNK // _L


def _sc_body(classes_hbm, logits_hbm, ema_hbm, out_hbm,
             xbuf, ema_v, logit_v, loss_v):
    wid = lax.axis_index("s") * _NC + lax.axis_index("c")
    base = wid * _ROWS_PER_TILE
    pltpu.sync_copy(ema_hbm, ema_v)
    row16 = lax.iota(jnp.int32, _L)
    for chunk in range(_NCHUNK):
        row0 = base + chunk * _CHUNK
        pltpu.sync_copy(classes_hbm.at[pl.ds(row0 * _C, _CHUNK * _C)], xbuf)
        pltpu.sync_copy(logits_hbm.at[pl.ds(row0, _CHUNK)], logit_v)
        for g in range(_GROUPS):
            rowbase = (row16 + (g * _L)) * _C

            def body(j, carry, rowbase=rowbase):
                best, bidx = carry
                v = plsc.load_gather(xbuf, [rowbase + j])
                pred = v > best
                best = jnp.where(pred, v, best)
                bidx = jnp.where(pred, jnp.full((_L,), j, dtype=jnp.int32),
                                 bidx)
                return best, bidx

            init = (jnp.full((_L,), -jnp.inf, jnp.float32),
                    jnp.zeros((_L,), jnp.int32))
            best, bidx = lax.fori_loop(0, _C, body, init)
            thr = plsc.load_gather(ema_v, [bidx])
            lg = logit_v[pl.ds(g * _L, _L)]
            d = jnp.maximum(thr - lg, 0.0)
            loss_v[pl.ds(g * _L, _L)] = d * d
        pltpu.sync_copy(loss_v, out_hbm.at[pl.ds(row0, _CHUNK)])


def kernel(gen_logits, gen_classes, ema_real):
    b, c = gen_classes.shape
    mesh = plsc.VectorSubcoreMesh(core_axis_name="c", subcore_axis_name="s")
    f = pl.kernel(
        _sc_body,
        out_type=jax.ShapeDtypeStruct((b,), jnp.float32),
        mesh=mesh,
        compiler_params=pltpu.CompilerParams(
            needs_layout_passes=False,
            use_tc_tiling_on_sc=False,
        ),
        scratch_types=[
            pltpu.VMEM((_CHUNK * c,), jnp.float32),
            pltpu.VMEM((c,), jnp.float32),
            pltpu.VMEM((_CHUNK,), jnp.float32),
            pltpu.VMEM((_CHUNK,), jnp.float32),
        ],
    )
    out = f(gen_classes.reshape(b * c), gen_logits.reshape(b), ema_real)
    return out.reshape(b, 1)


# speedup vs baseline: 1.9117x; 1.9117x over previous
import jax
import jax.numpy as jnp
from jax import lax
from jax.experimental import pallas as pl
from jax.experimental.pallas import tpu as pltpu
from jax.experimental.pallas import tpu_sc as plsc

_B = 16384
_C = 1000
_NC, _NS, _L = 2, 16, 16
_NW = _NC * _NS
_ROWS_PER_TILE = _B // _NW
_CHUNK = 32
_NCHUNK = _ROWS_PER_TILE // _CHUNK
_GROUPS = _CHUNK // _L
_UNROLL = 8
_STEPS = _C // _UNROLL


def _combine(aval, aidx, bval, bidx):
    pred = bval > aval
    return jnp.where(pred, bval, aval), jnp.where(pred, bidx, aidx)


def _sc_body(classes_hbm, logits_hbm, ema_hbm, out_hbm,
             xbuf0, xbuf1, ema_v, logit_v, loss_v, sem0, sem1):
    wid = lax.axis_index("s") * _NC + lax.axis_index("c")
    base = wid * _ROWS_PER_TILE
    pltpu.sync_copy(ema_hbm, ema_v)
    pltpu.sync_copy(logits_hbm.at[pl.ds(base, _ROWS_PER_TILE)], logit_v)
    row16 = lax.iota(jnp.int32, _L)

    bufs = (xbuf0, xbuf1)
    sems = (sem0, sem1)

    def chunk_dma(chunk):
        row0 = base + chunk * _CHUNK
        return pltpu.make_async_copy(
            classes_hbm.at[pl.ds(row0 * _C, _CHUNK * _C)],
            bufs[chunk % 2], sems[chunk % 2])

    chunk_dma(0).start()
    uconst = [jnp.full((_L,), u, dtype=jnp.int32) for u in range(_UNROLL)]

    for chunk in range(_NCHUNK):
        xbuf = bufs[chunk % 2]
        chunk_dma(chunk).wait()
        if chunk + 1 < _NCHUNK:
            chunk_dma(chunk + 1).start()
        for g in range(_GROUPS):
            rowbase = (row16 + (g * _L)) * _C

            def body(i, carry, rowbase=rowbase, xbuf=xbuf):
                best, bidx = carry
                j0 = i * _UNROLL
                vs = [plsc.load_gather(xbuf, [rowbase + (j0 + u)])
                      for u in range(_UNROLL)]
                val, idx = vs[0], uconst[0]
                pairs = [(vs[u], uconst[u]) for u in range(_UNROLL)]
                while len(pairs) > 1:
                    nxt = []
                    for k in range(0, len(pairs), 2):
                        nxt.append(_combine(pairs[k][0], pairs[k][1],
                                            pairs[k + 1][0], pairs[k + 1][1]))
                    pairs = nxt
                val, idx = pairs[0]
                jabs = jnp.full((_L,), j0, dtype=jnp.int32) + idx
                best, bidx = _combine(best, bidx, val, jabs)
                return best, bidx

            init = (jnp.full((_L,), -jnp.inf, jnp.float32),
                    jnp.zeros((_L,), jnp.int32))
            best, bidx = lax.fori_loop(0, _STEPS, body, init)
            thr = plsc.load_gather(ema_v, [bidx])
            lg = logit_v[pl.ds(chunk * _CHUNK + g * _L, _L)]
            d = jnp.maximum(thr - lg, 0.0)
            loss_v[pl.ds(chunk * _CHUNK + g * _L, _L)] = d * d
    pltpu.sync_copy(loss_v, out_hbm.at[pl.ds(base, _ROWS_PER_TILE)])


def kernel(gen_logits, gen_classes, ema_real):
    b, c = gen_classes.shape
    mesh = plsc.VectorSubcoreMesh(core_axis_name="c", subcore_axis_name="s")
    f = pl.kernel(
        _sc_body,
        out_type=jax.ShapeDtypeStruct((b,), jnp.float32),
        mesh=mesh,
        compiler_params=pltpu.CompilerParams(
            needs_layout_passes=False,
            use_tc_tiling_on_sc=False,
        ),
        scratch_types=[
            pltpu.VMEM((_CHUNK * c,), jnp.float32),
            pltpu.VMEM((_CHUNK * c,), jnp.float32),
            pltpu.VMEM((c,), jnp.float32),
            pltpu.VMEM((_ROWS_PER_TILE,), jnp.float32),
            pltpu.VMEM((_ROWS_PER_TILE,), jnp.float32),
            pltpu.SemaphoreType.DMA,
            pltpu.SemaphoreType.DMA,
        ],
    )
    out = f(gen_classes.reshape(b * c), gen_logits.reshape(b), ema_real)
    return out.reshape(b, 1)
